# baseline (speedup 1.0000x reference)
import jax
import jax.numpy as jnp
from jax.experimental import pallas as pl
from jax.experimental.pallas import tpu as pltpu

_BM = 400
_S = 10


def _make_body(bm, s, n, d, nb):
    rows = bm // s

    def _gcn_body(x_ref, adj_hbm, r_ref, w_ref, b_ref, out_ref, msg_ref,
                  acc0, acc1, abuf0, abuf1, sems):
        i = pl.program_id(0)

        def _issue(block, abuf, slot):
            for j in range(s):
                pltpu.make_async_copy(
                    adj_hbm.at[pl.ds(block * bm + j * rows, rows), :],
                    abuf.at[pl.ds(j * rows, rows), :],
                    sems.at[slot, j],
                ).start()

        def _wait(abuf, slot):
            for j in range(s):
                pltpu.make_async_copy(
                    adj_hbm.at[pl.ds(j * rows, rows), :],
                    abuf.at[pl.ds(j * rows, rows), :],
                    sems.at[slot, j],
                ).wait()

        @pl.when(i == 0)
        def _prime():
            _issue(0, abuf0, 0)

        def _epilogue(acc_ref):
            ax = (acc_ref[...] + r_ref[...]) * 0.5
            xi = x_ref[pl.ds((i - 1) * bm, bm), :]
            msg = jnp.dot(ax, w_ref[...], preferred_element_type=jnp.float32)
            ixw = jnp.dot(xi, w_ref[...], preferred_element_type=jnp.float32)
            msg_ref[...] = msg
            out_ref[...] = msg + ixw + b_ref[...]

        @pl.when(jnp.logical_and(i > 0, (i - 1) % 2 == 0))
        def _epi_even():
            _epilogue(acc0)

        @pl.when(jnp.logical_and(i > 0, (i - 1) % 2 == 1))
        def _epi_odd():
            _epilogue(acc1)

        def _compute(abuf, slot, acc_ref, nabuf, nslot):
            _wait(abuf, slot)

            @pl.when(i + 1 < nb)
            def _next():
                _issue(i + 1, nabuf, nslot)

            acc_ref[...] = jnp.dot(abuf[...], x_ref[...],
                                   preferred_element_type=jnp.float32)

        @pl.when(jnp.logical_and(i < nb, i % 2 == 0))
        def _even():
            _compute(abuf0, 0, acc0, abuf1, 1)

        @pl.when(jnp.logical_and(i < nb, i % 2 == 1))
        def _odd():
            _compute(abuf1, 1, acc1, abuf0, 0)

    return _gcn_body


def kernel(x, adj, AX_residual, weight, bias):
    n, d = x.shape
    bm, s = (_BM, _S) if n % _BM == 0 else (8, 1)
    nb = n // bm
    grid = (nb + 1,)

    def prev_block(i):
        return (jnp.maximum(i - 1, 0), 0)

    out_shape = [
        jax.ShapeDtypeStruct((n, d), jnp.float32),
        jax.ShapeDtypeStruct((n, d), jnp.float32),
    ]
    out, msg = pl.pallas_call(
        _make_body(bm, s, n, d, nb),
        grid=grid,
        in_specs=[
            pl.BlockSpec(memory_space=pltpu.VMEM),
            pl.BlockSpec(memory_space=pl.ANY),
            pl.BlockSpec((bm, d), prev_block),
            pl.BlockSpec(memory_space=pltpu.VMEM),
            pl.BlockSpec(memory_space=pltpu.VMEM),
        ],
        out_specs=[
            pl.BlockSpec((bm, d), prev_block),
            pl.BlockSpec((bm, d), prev_block),
        ],
        out_shape=out_shape,
        scratch_shapes=[
            pltpu.VMEM((bm, d), jnp.float32),
            pltpu.VMEM((bm, d), jnp.float32),
            pltpu.VMEM((bm, n), jnp.float32),
            pltpu.VMEM((bm, n), jnp.float32),
            pltpu.SemaphoreType.DMA((2, s)),
        ],
        compiler_params=pltpu.CompilerParams(
            dimension_semantics=("arbitrary",),
            vmem_limit_bytes=100 * 1024 * 1024,
        ),
    )(x, adj, AX_residual, weight, bias.reshape(1, d))
    return out, msg


# speedup vs baseline: 1.0276x; 1.0276x over previous
import jax
import jax.numpy as jnp
from jax.experimental import pallas as pl
from jax.experimental.pallas import tpu as pltpu

_BM = 400
_S = 10


def _make_body(bm, s, n, d, nb):
    rows = bm // s

    def _gcn_body(x_hbm, adj_hbm, r_ref, w_ref, b_ref, out_ref, msg_ref,
                  acc0, acc1, abuf0, abuf1, x_ref, sems, xsems):
        i = pl.program_id(0)
        xrows = n // s

        def _issue_x():
            for j in range(s):
                pltpu.make_async_copy(
                    x_hbm.at[pl.ds(j * xrows, xrows), :],
                    x_ref.at[pl.ds(j * xrows, xrows), :],
                    xsems.at[j],
                ).start()

        def _wait_x():
            for j in range(s):
                pltpu.make_async_copy(
                    x_hbm.at[pl.ds(j * xrows, xrows), :],
                    x_ref.at[pl.ds(j * xrows, xrows), :],
                    xsems.at[j],
                ).wait()

        def _issue(block, abuf, slot):
            for j in range(s):
                pltpu.make_async_copy(
                    adj_hbm.at[pl.ds(block * bm + j * rows, rows), :],
                    abuf.at[pl.ds(j * rows, rows), :],
                    sems.at[slot, j],
                ).start()

        def _wait(abuf, slot):
            for j in range(s):
                pltpu.make_async_copy(
                    adj_hbm.at[pl.ds(j * rows, rows), :],
                    abuf.at[pl.ds(j * rows, rows), :],
                    sems.at[slot, j],
                ).wait()

        @pl.when(i == 0)
        def _prime():
            _issue(0, abuf0, 0)
            _issue_x()

        @pl.when(jnp.logical_and(i + 1 < nb, (i + 1) % 2 == 0))
        def _next_even():
            _issue(i + 1, abuf0, 0)

        @pl.when(jnp.logical_and(i + 1 < nb, (i + 1) % 2 == 1))
        def _next_odd():
            _issue(i + 1, abuf1, 1)

        def _epilogue(acc_ref):
            ax = (acc_ref[...] + r_ref[...]) * 0.5
            xi = x_ref[pl.ds((i - 1) * bm, bm), :]
            msg = jnp.dot(ax, w_ref[...], preferred_element_type=jnp.float32)
            ixw = jnp.dot(xi, w_ref[...], preferred_element_type=jnp.float32)
            msg_ref[...] = msg
            out_ref[...] = msg + ixw + b_ref[...]

        @pl.when(jnp.logical_and(i > 0, (i - 1) % 2 == 0))
        def _epi_even():
            _epilogue(acc0)

        @pl.when(jnp.logical_and(i > 0, (i - 1) % 2 == 1))
        def _epi_odd():
            _epilogue(acc1)

        def _compute(abuf, slot, acc_ref):
            _wait(abuf, slot)

            @pl.when(i == 0)
            def _wx():
                _wait_x()
            acc_ref[...] = jnp.dot(abuf[...], x_ref[...],
                                   preferred_element_type=jnp.float32)

        @pl.when(jnp.logical_and(i < nb, i % 2 == 0))
        def _even():
            _compute(abuf0, 0, acc0)

        @pl.when(jnp.logical_and(i < nb, i % 2 == 1))
        def _odd():
            _compute(abuf1, 1, acc1)

    return _gcn_body


def kernel(x, adj, AX_residual, weight, bias):
    n, d = x.shape
    bm, s = (_BM, _S) if n % _BM == 0 else (8, 1)
    nb = n // bm
    grid = (nb + 1,)

    def prev_block(i):
        return (jnp.maximum(i - 1, 0), 0)

    out_shape = [
        jax.ShapeDtypeStruct((n, d), jnp.float32),
        jax.ShapeDtypeStruct((n, d), jnp.float32),
    ]
    out, msg = pl.pallas_call(
        _make_body(bm, s, n, d, nb),
        grid=grid,
        in_specs=[
            pl.BlockSpec(memory_space=pl.ANY),
            pl.BlockSpec(memory_space=pl.ANY),
            pl.BlockSpec((bm, d), prev_block),
            pl.BlockSpec(memory_space=pltpu.VMEM),
            pl.BlockSpec(memory_space=pltpu.VMEM),
        ],
        out_specs=[
            pl.BlockSpec((bm, d), prev_block),
            pl.BlockSpec((bm, d), prev_block),
        ],
        out_shape=out_shape,
        scratch_shapes=[
            pltpu.VMEM((bm, d), jnp.float32),
            pltpu.VMEM((bm, d), jnp.float32),
            pltpu.VMEM((bm, n), jnp.float32),
            pltpu.VMEM((bm, n), jnp.float32),
            pltpu.VMEM((n, d), jnp.float32),
            pltpu.SemaphoreType.DMA((2, s)),
            pltpu.SemaphoreType.DMA((s,)),
        ],
        compiler_params=pltpu.CompilerParams(
            dimension_semantics=("arbitrary",),
            vmem_limit_bytes=100 * 1024 * 1024,
        ),
    )(x, adj, AX_residual, weight, bias.reshape(1, d))
    return out, msg


# speedup vs baseline: 1.0626x; 1.0340x over previous
import jax
import jax.numpy as jnp
from jax.experimental import pallas as pl
from jax.experimental.pallas import tpu as pltpu

_BM = 400
_S = 10


def _make_body(bm, s, n, d, nb):
    rows = bm // s

    def _gcn_body(x_ref, adj_hbm, r_ref, w_ref, b_ref, out_ref, msg_ref,
                  acc0, acc1, abuf0, abuf1, sems):
        i = pl.program_id(0)

        def _issue(block, abuf, slot):
            for j in range(s):
                pltpu.make_async_copy(
                    adj_hbm.at[pl.ds(block * bm + j * rows, rows), :],
                    abuf.at[pl.ds(j * rows, rows), :],
                    sems.at[slot, j],
                ).start()

        def _wait(abuf, slot):
            for j in range(s):
                pltpu.make_async_copy(
                    adj_hbm.at[pl.ds(j * rows, rows), :],
                    abuf.at[pl.ds(j * rows, rows), :],
                    sems.at[slot, j],
                ).wait()

        @pl.when(i == 0)
        def _prime():
            _issue(0, abuf0, 0)

        @pl.when(jnp.logical_and(i + 1 < nb, (i + 1) % 2 == 0))
        def _next_even():
            _issue(i + 1, abuf0, 0)

        @pl.when(jnp.logical_and(i + 1 < nb, (i + 1) % 2 == 1))
        def _next_odd():
            _issue(i + 1, abuf1, 1)

        def _epilogue(acc_ref):
            ax = (acc_ref[...] + r_ref[...]) * 0.5
            xi = x_ref[pl.ds((i - 1) * bm, bm), :]
            msg = jnp.dot(ax, w_ref[...], preferred_element_type=jnp.float32)
            ixw = jnp.dot(xi, w_ref[...], preferred_element_type=jnp.float32)
            msg_ref[...] = msg
            out_ref[...] = msg + ixw + b_ref[...]

        @pl.when(jnp.logical_and(i > 0, (i - 1) % 2 == 0))
        def _epi_even():
            _epilogue(acc0)

        @pl.when(jnp.logical_and(i > 0, (i - 1) % 2 == 1))
        def _epi_odd():
            _epilogue(acc1)

        def _compute(abuf, slot, acc_ref):
            _wait(abuf, slot)
            acc_ref[...] = jnp.dot(abuf[...], x_ref[...],
                                   preferred_element_type=jnp.float32)

        @pl.when(jnp.logical_and(i < nb, i % 2 == 0))
        def _even():
            _compute(abuf0, 0, acc0)

        @pl.when(jnp.logical_and(i < nb, i % 2 == 1))
        def _odd():
            _compute(abuf1, 1, acc1)

    return _gcn_body


def kernel(x, adj, AX_residual, weight, bias):
    n, d = x.shape
    bm, s = (_BM, _S) if n % _BM == 0 else (8, 1)
    nb = n // bm
    grid = (nb + 1,)

    def prev_block(i):
        return (jnp.maximum(i - 1, 0), 0)

    out_shape = [
        jax.ShapeDtypeStruct((n, d), jnp.float32),
        jax.ShapeDtypeStruct((n, d), jnp.float32),
    ]
    out, msg = pl.pallas_call(
        _make_body(bm, s, n, d, nb),
        grid=grid,
        in_specs=[
            pl.BlockSpec(memory_space=pltpu.VMEM),
            pl.BlockSpec(memory_space=pl.ANY),
            pl.BlockSpec((bm, d), prev_block),
            pl.BlockSpec(memory_space=pltpu.VMEM),
            pl.BlockSpec(memory_space=pltpu.VMEM),
        ],
        out_specs=[
            pl.BlockSpec((bm, d), prev_block),
            pl.BlockSpec((bm, d), prev_block),
        ],
        out_shape=out_shape,
        scratch_shapes=[
            pltpu.VMEM((bm, d), jnp.float32),
            pltpu.VMEM((bm, d), jnp.float32),
            pltpu.VMEM((bm, n), jnp.float32),
            pltpu.VMEM((bm, n), jnp.float32),
            pltpu.SemaphoreType.DMA((2, s)),
        ],
        compiler_params=pltpu.CompilerParams(
            dimension_semantics=("arbitrary",),
            vmem_limit_bytes=100 * 1024 * 1024,
        ),
    )(x, adj, AX_residual, weight, bias.reshape(1, d))
    return out, msg
